# baseline (speedup 1.0000x reference)
import functools

import jax
import jax.numpy as jnp
from jax import lax
from jax.experimental import pallas as pl
from jax.experimental.pallas import tpu as pltpu
from jax.experimental.pallas import tpu_sc as plsc

VOCAB = 1000
DIM = 256
DSPLIT = 4
DSUB = DIM // DSPLIT
CHUNK = 128
IDX_BLOCK = 32
NBUF = 2


@functools.cache
def _build(B):
    info = plsc.get_sparse_core_info()
    NC, NS = info.num_cores, info.num_subcores
    NW = NC * NS
    NG = NW // DSPLIT
    b_per_g = B // NG
    assert b_per_g * NG == B and b_per_g % (CHUNK * IDX_BLOCK) == 0
    n_blocks = b_per_g // (CHUNK * IDX_BLOCK)
    n_chunks = b_per_g // CHUNK
    mesh = plsc.VectorSubcoreMesh(core_axis_name="c", subcore_axis_name="s")

    @functools.partial(
        pl.kernel,
        mesh=mesh,
        out_type=jax.ShapeDtypeStruct((B, DIM), jnp.float32),
        compiler_params=pltpu.CompilerParams(use_tc_tiling_on_sc=False,
                                             needs_layout_passes=False),
        scratch_types=[
            pltpu.VMEM((VOCAB, DSUB), jnp.float32),
            pltpu.VMEM((3, IDX_BLOCK * CHUNK), jnp.int32),
            pltpu.VMEM((NBUF * CHUNK, DSUB), jnp.float32),
            pltpu.SemaphoreType.DMA((3,)),
            pltpu.SemaphoreType.DMA((NBUF,)),
            pltpu.SemaphoreType.DMA,
        ],
    )
    def lookup(table_hbm, idx_hbm, out_hbm, tab_v, idx_v, rows_v,
               isem, wsem, tsem):
        wid = lax.axis_index("s") * NC + lax.axis_index("c")
        grp = wid // DSPLIT
        dpart = wid % DSPLIT
        d0 = dpart * DSUB
        base = grp * b_per_g

        def stage(ob, slot):
            pltpu.async_copy(idx_hbm.at[grp, ob], idx_v.at[slot],
                             isem.at[slot])

        def wait_idx(slot):
            pltpu.make_async_copy(idx_hbm.at[0, 0], idx_v.at[slot],
                                  isem.at[slot]).wait()

        def fire_write(pos, buf):
            pltpu.async_copy(rows_v.at[pl.ds(buf * CHUNK, CHUNK)],
                             out_hbm.at[pl.ds(pos, CHUNK), pl.ds(d0, DSUB)],
                             wsem.at[buf])

        def wait_write(buf):
            pltpu.make_async_copy(rows_v.at[pl.ds(0, CHUNK)],
                                  out_hbm.at[pl.ds(0, CHUNK),
                                             pl.ds(0, DSUB)],
                                  wsem.at[buf]).wait()

        pltpu.async_copy(table_hbm.at[:, pl.ds(d0, DSUB)], tab_v, tsem)
        stage(0, 0)
        stage(1, 1)
        stage(2, 2)
        pltpu.make_async_copy(table_hbm.at[:, pl.ds(0, DSUB)], tab_v,
                              tsem).wait()

        qrows = [lax.iota(jnp.int32, 16) + 16 * q
                 for q in range(CHUNK // 16)]

        def chunk_body(g, carry):
            slot = (g // IDX_BLOCK) % 3
            j = g % IDX_BLOCK
            buf = g % NBUF
            rbase = buf * CHUNK

            @pl.when(g >= NBUF)
            def _():
                wait_write(buf)

            nq = CHUNK // 16
            ivs = [idx_v[slot, pl.ds(j * CHUNK + 16 * q, 16)]
                   for q in range(nq)]
            rqs = [qrows[q] + rbase for q in range(nq)]
            for d in range(DSUB):
                dv = jnp.full((16,), d, jnp.int32)
                vals = [plsc.load_gather(tab_v, [ivs[q], dv])
                        for q in range(nq)]
                for q in range(nq):
                    plsc.store_scatter(rows_v, [rqs[q], dv], vals[q])
            fire_write(base + g * CHUNK, buf)

            @pl.when(j == IDX_BLOCK - 1)
            def _():
                ob = g // IDX_BLOCK

                @pl.when(ob + 3 < n_blocks)
                def _():
                    stage(ob + 3, ob % 3)

                @pl.when(ob + 1 < n_blocks)
                def _():
                    wait_idx((ob + 1) % 3)

            return carry

        wait_idx(0)
        lax.fori_loop(0, n_chunks, chunk_body, 0, unroll=False)

        for k in range(NBUF):
            wait_write((n_chunks - 1 - k) % NBUF)

    def run(table, idx_flat):
        idx3 = idx_flat.reshape(NG, n_blocks, IDX_BLOCK * CHUNK)
        return lookup(table, idx3)

    return run


def kernel(x, emb_weight):
    b, s = x.shape
    idx = x.reshape(-1).astype(jnp.int32)
    out = _build(idx.shape[0])(emb_weight, idx)
    return out.reshape(b, s, DIM)


# speedup vs baseline: 6.1080x; 6.1080x over previous
import functools

import jax
import jax.numpy as jnp
from jax import lax
from jax.experimental import pallas as pl
from jax.experimental.pallas import tpu as pltpu
from jax.experimental.pallas import tpu_sc as plsc

VOCAB = 1000
DIM = 256
DSPLIT = 4
DSUB = DIM // DSPLIT
CHUNK = 128
IDX_BLOCK = 32
NBUF = 2


@functools.cache
def _build(B):
    info = plsc.get_sparse_core_info()
    NC, NS = info.num_cores, info.num_subcores
    NW = NC * NS
    NG = NW // DSPLIT
    b_per_g = B // NG
    assert b_per_g * NG == B and b_per_g % (CHUNK * IDX_BLOCK) == 0
    n_blocks = b_per_g // (CHUNK * IDX_BLOCK)
    n_chunks = b_per_g // CHUNK
    mesh = plsc.VectorSubcoreMesh(core_axis_name="c", subcore_axis_name="s")

    @functools.partial(
        pl.kernel,
        mesh=mesh,
        out_type=jax.ShapeDtypeStruct((B, DIM), jnp.float32),
        compiler_params=pltpu.CompilerParams(use_tc_tiling_on_sc=False,
                                             needs_layout_passes=False),
        scratch_types=[
            pltpu.VMEM((VOCAB, DSUB), jnp.float32),
            pltpu.VMEM((3, IDX_BLOCK * CHUNK), jnp.int32),
            pltpu.VMEM((NBUF * CHUNK, DSUB), jnp.float32),
            pltpu.SemaphoreType.DMA((3,)),
            pltpu.SemaphoreType.DMA((NBUF,)),
            pltpu.SemaphoreType.DMA,
        ],
    )
    def lookup(table_hbm, idx_hbm, out_hbm, tab_v, idx_v, rows_v,
               isem, wsem, tsem):
        wid = lax.axis_index("s") * NC + lax.axis_index("c")
        grp = wid // DSPLIT
        dpart = wid % DSPLIT
        d0 = dpart * DSUB
        base = grp * b_per_g

        def stage(ob, slot):
            pltpu.async_copy(idx_hbm.at[grp, ob], idx_v.at[slot],
                             isem.at[slot])

        def wait_idx(slot):
            pltpu.make_async_copy(idx_hbm.at[0, 0], idx_v.at[slot],
                                  isem.at[slot]).wait()

        def fire_write(pos, buf):
            pltpu.async_copy(rows_v.at[pl.ds(buf * CHUNK, CHUNK)],
                             out_hbm.at[pl.ds(pos, CHUNK), pl.ds(d0, DSUB)],
                             wsem.at[buf])

        def wait_write(buf):
            pltpu.make_async_copy(rows_v.at[pl.ds(0, CHUNK)],
                                  out_hbm.at[pl.ds(0, CHUNK),
                                             pl.ds(0, DSUB)],
                                  wsem.at[buf]).wait()

        pltpu.async_copy(table_hbm.at[:, pl.ds(d0, DSUB)], tab_v, tsem)
        stage(0, 0)
        stage(1, 1)
        stage(2, 2)
        pltpu.make_async_copy(table_hbm.at[:, pl.ds(0, DSUB)], tab_v,
                              tsem).wait()

        lanes = lax.iota(jnp.int32, 16)
        gdn = lax.GatherDimensionNumbers(
            offset_dims=(), collapsed_slice_dims=(0,), start_index_map=(0,))
        lane_ids = [jnp.full((16, 1), l, jnp.int32) for l in range(16)]
        colvs = [lanes + 16 * k for k in range(DSUB // 16)]
        nk = DSUB // 16

        def bcast(v, l):
            return lax.gather(v, lane_ids[l], gdn, (1,),
                              mode=lax.GatherScatterMode.PROMISE_IN_BOUNDS)

        def chunk_body(g, carry):
            slot = (g // IDX_BLOCK) % 3
            j = g % IDX_BLOCK
            buf = g % NBUF
            rbase = buf * CHUNK

            @pl.when(g >= NBUF)
            def _():
                wait_write(buf)

            for q in range(CHUNK // 16):
                iv = idx_v[slot, pl.ds(j * CHUNK + 16 * q, 16)]
                rss = [bcast(iv, l) for l in range(16)]
                prev = None
                for l in range(16):
                    vals = [plsc.load_gather(tab_v, [rss[l], colvs[k]])
                            for k in range(nk)]
                    if prev is not None:
                        pu, pvals = prev
                        for k in range(nk):
                            rows_v[pu, pl.ds(16 * k, 16)] = pvals[k]
                    prev = (rbase + 16 * q + l, vals)
                pu, pvals = prev
                for k in range(nk):
                    rows_v[pu, pl.ds(16 * k, 16)] = pvals[k]
            fire_write(base + g * CHUNK, buf)

            @pl.when(j == IDX_BLOCK - 1)
            def _():
                ob = g // IDX_BLOCK

                @pl.when(ob + 3 < n_blocks)
                def _():
                    stage(ob + 3, ob % 3)

                @pl.when(ob + 1 < n_blocks)
                def _():
                    wait_idx((ob + 1) % 3)

            return carry

        wait_idx(0)
        lax.fori_loop(0, n_chunks, chunk_body, 0, unroll=False)

        for k in range(NBUF):
            wait_write((n_chunks - 1 - k) % NBUF)

    def run(table, idx_flat):
        idx3 = idx_flat.reshape(NG, n_blocks, IDX_BLOCK * CHUNK)
        return lookup(table, idx3)

    return run


def kernel(x, emb_weight):
    b, s = x.shape
    idx = x.reshape(-1).astype(jnp.int32)
    out = _build(idx.shape[0])(emb_weight, idx)
    return out.reshape(b, s, DIM)
